# baseline (speedup 1.0000x reference)
import jax
import jax.numpy as jnp
from jax.experimental import pallas as pl
from jax.experimental.pallas import tpu as pltpu

N_EMB = 1024
EMB_DIM = 32
N_TOK = 65536

TILE = 1024
GRID = N_TOK // TILE


def _vq_tile(x_ref, lab_ref, w_ref, enc_ref, quant_ref, idx_ref, acc_ref):
    i = pl.program_id(0)

    @pl.when(i == 0)
    def _():
        acc_ref[...] = jnp.zeros((1, 1), jnp.float32)

    x = x_ref[...]
    xn = x / jnp.maximum(
        jnp.sqrt(jnp.sum(x * x, axis=1, keepdims=True)), 1e-12)
    w = w_ref[...]
    wn = w / jnp.maximum(
        jnp.sqrt(jnp.sum(w * w, axis=1, keepdims=True)), 1e-12)

    dT = jnp.dot(wn, xn.T, preferred_element_type=jnp.float32)

    idx = jnp.argmax(dT, axis=0).astype(jnp.int32)

    row = jax.lax.broadcasted_iota(jnp.int32, (N_EMB, TILE), 0)
    lab = lab_ref[...]
    gathered = jnp.sum(jnp.where(row == lab[None, :], dT, 0.0), axis=0)
    acc_ref[...] += jnp.sum(gathered).reshape(1, 1)

    col = jax.lax.broadcasted_iota(jnp.int32, (TILE, N_EMB), 1)
    onehot = (col == idx[:, None]).astype(jnp.float32)
    enc_ref[...] = onehot
    quant_ref[...] = jnp.dot(onehot, w, preferred_element_type=jnp.float32)
    idx_ref[...] = idx


def kernel(inputs, labels, W):
    input_shape = inputs.shape
    flat = inputs.reshape(-1, EMB_DIM)
    lab = labels.astype(jnp.int32)

    enc, quant, idx, acc = pl.pallas_call(
        _vq_tile,
        grid=(GRID,),
        in_specs=[
            pl.BlockSpec((TILE, EMB_DIM), lambda i: (i, 0)),
            pl.BlockSpec((TILE,), lambda i: (i,)),
            pl.BlockSpec((N_EMB, EMB_DIM), lambda i: (0, 0)),
        ],
        out_specs=[
            pl.BlockSpec((TILE, N_EMB), lambda i: (i, 0)),
            pl.BlockSpec((TILE, EMB_DIM), lambda i: (i, 0)),
            pl.BlockSpec((TILE,), lambda i: (i,)),
            pl.BlockSpec((1, 1), lambda i: (0, 0)),
        ],
        out_shape=[
            jax.ShapeDtypeStruct((N_TOK, N_EMB), jnp.float32),
            jax.ShapeDtypeStruct((N_TOK, EMB_DIM), jnp.float32),
            jax.ShapeDtypeStruct((N_TOK,), jnp.int32),
            jax.ShapeDtypeStruct((1, 1), jnp.float32),
        ],
    )(flat, lab, W)

    loss = (1.0 - acc[0, 0] / N_TOK).astype(jnp.float32)
    quantized = quant.reshape(input_shape)
    perplexity = jnp.array(1, dtype=jnp.int32)
    encoding_indices = idx[:, None]
    return (loss, quantized, perplexity, enc, encoding_indices)


# speedup vs baseline: 1.0541x; 1.0541x over previous
import functools

import jax
import jax.numpy as jnp
from jax import lax
from jax.experimental import pallas as pl
from jax.experimental.pallas import tpu as pltpu
from jax.experimental.pallas import tpu_sc as plsc

N_EMB = 1024
EMB_DIM = 32
N_TOK = 65536

TILE = 2048
GRID = N_TOK // TILE

NC = 2
NS = 16
LANES = 16
NW = NC * NS
SC_CHUNK = 512
TOK_PER_W = N_TOK // NW


def _vq_tile(x_ref, w_ref, enc_ref, quant_ref, idx_ref):
    x = x_ref[...]
    xn = x / jnp.maximum(
        jnp.sqrt(jnp.sum(x * x, axis=1, keepdims=True)), 1e-12)
    w = w_ref[...]
    wn = w / jnp.maximum(
        jnp.sqrt(jnp.sum(w * w, axis=1, keepdims=True)), 1e-12)

    dT = jnp.dot(wn, xn.T, preferred_element_type=jnp.float32)

    idx = jnp.argmax(dT, axis=0).astype(jnp.int32)

    col = jax.lax.broadcasted_iota(jnp.int32, (TILE, N_EMB), 1)
    onehot = (col == idx[:, None]).astype(jnp.float32)
    enc_ref[...] = onehot
    quant_ref[...] = jnp.dot(onehot, w, preferred_element_type=jnp.float32)
    idx_ref[...] = idx


def _newton_rsqrt(s):
    i = plsc.bitcast(s, jnp.int32)
    i = jnp.full((LANES,), 0x5F3759DF, jnp.int32) - jnp.right_shift(
        i, jnp.full((LANES,), 1, jnp.int32))
    y = plsc.bitcast(i, jnp.float32)
    for _ in range(4):
        y = y * (1.5 - 0.5 * s * y * y)
    return y


def _sc_loss_body(x_hbm, lab_hbm, w_hbm, out_hbm, lab_v, x_v, w_v,
                  stage_v, sem):
    wid = lax.axis_index("s") * NC + lax.axis_index("c")
    base = wid * TOK_PER_W
    pltpu.sync_copy(w_hbm, w_v)
    lane_iota = lax.iota(jnp.int32, LANES)
    total = jnp.zeros((LANES,), jnp.float32)

    for c in range(TOK_PER_W // SC_CHUNK):
        cbase = base + c * SC_CHUNK
        pltpu.sync_copy(lab_hbm.at[pl.ds(cbase, SC_CHUNK)], lab_v)
        pltpu.sync_copy(
            x_hbm.at[pl.ds(cbase * EMB_DIM, SC_CHUNK * EMB_DIM)], x_v)

        def body(j, tot):
            tok = j * LANES + lane_iota
            lv = plsc.load_gather(lab_v, [tok])
            xoff = tok * EMB_DIM
            woff = lv * EMB_DIM
            dot = jnp.zeros((LANES,), jnp.float32)
            nx2 = jnp.zeros((LANES,), jnp.float32)
            nw2 = jnp.zeros((LANES,), jnp.float32)
            for k in range(EMB_DIM):
                xk = plsc.load_gather(x_v, [xoff + k])
                wk = plsc.load_gather(w_v, [woff + k])
                dot = dot + xk * wk
                nx2 = nx2 + xk * xk
                nw2 = nw2 + wk * wk
            s = jnp.maximum(nx2, 1e-24) * jnp.maximum(nw2, 1e-24)
            return tot + dot * _newton_rsqrt(s)

        total = lax.fori_loop(0, SC_CHUNK // LANES, body, total)

    stage_v[...] = total
    pltpu.sync_copy(stage_v, out_hbm.at[pl.ds(wid * LANES, LANES)])


def _sc_loss_partials(flat1d, lab, w1d):
    mesh = plsc.VectorSubcoreMesh(core_axis_name="c", subcore_axis_name="s")
    k = functools.partial(
        pl.kernel,
        out_type=jax.ShapeDtypeStruct((NW * LANES,), jnp.float32),
        mesh=mesh,
        compiler_params=pltpu.CompilerParams(needs_layout_passes=False),
        scratch_types=[
            pltpu.VMEM((SC_CHUNK,), jnp.int32),
            pltpu.VMEM((SC_CHUNK * EMB_DIM,), jnp.float32),
            pltpu.VMEM((N_EMB * EMB_DIM,), jnp.float32),
            pltpu.VMEM((LANES,), jnp.float32),
            pltpu.SemaphoreType.DMA,
        ],
    )(_sc_loss_body)
    return k(flat1d, lab, w1d)


def kernel(inputs, labels, W):
    input_shape = inputs.shape
    flat = inputs.reshape(-1, EMB_DIM)
    lab = labels.astype(jnp.int32)

    partials = _sc_loss_partials(flat.reshape(-1), lab, W.reshape(-1))

    enc, quant, idx = pl.pallas_call(
        _vq_tile,
        grid=(GRID,),
        in_specs=[
            pl.BlockSpec((TILE, EMB_DIM), lambda i: (i, 0)),
            pl.BlockSpec((N_EMB, EMB_DIM), lambda i: (0, 0)),
        ],
        out_specs=[
            pl.BlockSpec((TILE, N_EMB), lambda i: (i, 0)),
            pl.BlockSpec((TILE, EMB_DIM), lambda i: (i, 0)),
            pl.BlockSpec((TILE,), lambda i: (i,)),
        ],
        out_shape=[
            jax.ShapeDtypeStruct((N_TOK, N_EMB), jnp.float32),
            jax.ShapeDtypeStruct((N_TOK, EMB_DIM), jnp.float32),
            jax.ShapeDtypeStruct((N_TOK,), jnp.int32),
        ],
    )(flat, W)

    loss = (1.0 - jnp.sum(partials) / N_TOK).astype(jnp.float32)
    quantized = quant.reshape(input_shape)
    perplexity = jnp.array(1, dtype=jnp.int32)
    encoding_indices = idx[:, None]
    return (loss, quantized, perplexity, enc, encoding_indices)
